# baseline (speedup 1.0000x reference)
import functools

import jax
import jax.numpy as jnp
from jax import lax
from jax.experimental import pallas as pl
from jax.experimental.pallas import tpu as pltpu
from jax.experimental.pallas import tpu_sc as plsc

N = 10000
E = 160000
D = 256
HALF = 128
OUT_D = 3

NCORES = 2
NSUB = 16
EPT = E // NSUB
CB = 125
NCHUNK = EPT // CB
CNT_CB = 80
ZROWS = 40
NZB = N // ZROWS
NGRP = 2
GCH = NCHUNK // NGRP

BR = 1000
GRID = N // BR


def _bdot(x, w):
    return jnp.dot(x.astype(jnp.bfloat16), w, preferred_element_type=jnp.float32)


def _msg(x, w1, b1, w2, b2):
    h = jnp.maximum(_bdot(x, w1) + b1, 0.0)
    return _bdot(h, w2) + b2


def _enc_body(x_ref, we1, be1, we2, be2, we3, be3, g_ref, b_ref,
              w1a, b1e, w2, b2, x0_ref, ml_ref, mr_ref):
    h = jnp.maximum(_bdot(x_ref[...], we1[...]) + be1[...], 0.0)
    h = jnp.maximum(_bdot(h, we2[...]) + be2[...], 0.0)
    h = _bdot(h, we3[...]) + be3[...]
    mu = jnp.mean(h, axis=1, keepdims=True)
    var = jnp.mean((h - mu) * (h - mu), axis=1, keepdims=True)
    x0 = (h - mu) / jnp.sqrt(var + 1e-5) * g_ref[...] + b_ref[...]
    x0_ref[...] = x0
    mm = _msg(x0, w1a[...], b1e[...], w2[...], b2[...])
    ml_ref[...] = mm[:, :HALF]
    mr_ref[...] = mm[:, HALF:]


def _step_body(x_ref, sl_ref, sr_ref, r_ref, ws, bs,
               w1a, b1e, w2, b2, xt_ref, ml_ref, mr_ref):
    aggr = jnp.concatenate([sl_ref[...], sr_ref[...]], axis=1) * r_ref[...]
    xt = _bdot(x_ref[...], ws[...]) + bs[...] + aggr
    xt_ref[...] = xt
    mm = _msg(xt, w1a[...], b1e[...], w2[...], b2[...])
    ml_ref[...] = mm[:, :HALF]
    mr_ref[...] = mm[:, HALF:]


def _last_body(x_ref, sl_ref, sr_ref, r_ref, ws, bs,
               wd1, bd1, wd2, bd2, wd3, bd3, o_ref):
    aggr = jnp.concatenate([sl_ref[...], sr_ref[...]], axis=1) * r_ref[...]
    xt = _bdot(x_ref[...], ws[...]) + bs[...] + aggr
    h = jnp.maximum(_bdot(xt, wd1[...]) + bd1[...], 0.0)
    h = jnp.maximum(_bdot(h, wd2[...]) + bd2[...], 0.0)
    o_ref[...] = _bdot(h, wd3[...]) + bd3[...]


def _row_spec(width):
    return pl.BlockSpec((BR, width), lambda i: (i, 0))


def _full_spec(shape):
    return pl.BlockSpec(shape, lambda i: tuple(0 for _ in shape))


def _wspec(a):
    return _full_spec(a.shape)


def _f32(shape):
    return jax.ShapeDtypeStruct(shape, jnp.float32)


def _enc_call(x, weights):
    in_specs = [_row_spec(D)] + [_wspec(w) for w in weights]
    return pl.pallas_call(
        _enc_body,
        grid=(GRID,),
        in_specs=in_specs,
        out_specs=[_row_spec(D), _row_spec(HALF), _row_spec(HALF)],
        out_shape=[_f32((N, D)), _f32((N, HALF)), _f32((N, HALF))],
    )(x, *weights)


def _step_call(x, sl, sr, recip, weights):
    in_specs = [_row_spec(D), _row_spec(HALF), _row_spec(HALF), _row_spec(1)]
    in_specs += [_wspec(w) for w in weights]
    return pl.pallas_call(
        _step_body,
        grid=(GRID,),
        in_specs=in_specs,
        out_specs=[_row_spec(D), _row_spec(HALF), _row_spec(HALF)],
        out_shape=[_f32((N, D)), _f32((N, HALF)), _f32((N, HALF))],
    )(x, sl, sr, recip, *weights)


def _last_call(x, sl, sr, recip, weights):
    in_specs = [_row_spec(D), _row_spec(HALF), _row_spec(HALF), _row_spec(1)]
    in_specs += [_wspec(w) for w in weights]
    return pl.pallas_call(
        _last_body,
        grid=(GRID,),
        in_specs=in_specs,
        out_specs=[_row_spec(OUT_D)],
        out_shape=[_f32((N, OUT_D))],
    )(x, sl, sr, recip, *weights)[0]


_MESH = plsc.VectorSubcoreMesh(
    core_axis_name="c", subcore_axis_name="s",
    num_cores=NCORES, num_subcores=NSUB)

_CNT_BLK = N // CNT_CB


def _half_pipeline(sid, m_hbm, src3, dst3, out_hbm, srcv, dstv, rows0, rows1,
                   zbuf, acc, sem0, sem1, cnt_parts):
    nz = jnp.where(sid < NZB % NSUB, NZB // NSUB + 1, NZB // NSUB)

    def zbody(i, _):
        blk = sid + i * NSUB
        pltpu.sync_copy(zbuf, acc.at[pl.ds(blk * ZROWS, ZROWS)])
        return ()
    lax.fori_loop(0, nz, zbody, (), unroll=False)

    if cnt_parts is not None:
        onesv, zc, acc_cnt, cnt_out, cwb = cnt_parts
        ncz = jnp.where(sid < _CNT_BLK % NSUB, _CNT_BLK // NSUB + 1,
                        _CNT_BLK // NSUB)

        def czbody(i, _):
            blk = sid + i * NSUB
            pltpu.sync_copy(zc, acc_cnt.at[pl.ds(blk * CNT_CB, CNT_CB)])
            return ()
        lax.fori_loop(0, ncz, czbody, (), unroll=False)

    plsc.subcore_barrier()

    def fire(j, buf, sem):
        pltpu.async_copy(m_hbm.at[srcv.at[j]], buf, sem)

    def wait(buf, sem):
        pltpu.make_async_copy(m_hbm.at[srcv.at[0]], buf, sem).wait()

    def scat(j, buf):
        pltpu.sync_copy(buf, acc.at[dstv.at[j]], add=True)
        if cnt_parts is not None:
            pltpu.sync_copy(cnt_parts[0], cnt_parts[2].at[dstv.at[j]],
                            add=True)

    for g in range(NGRP):
        pltpu.sync_copy(src3.at[sid, pl.ds(g * GCH, GCH)], srcv)
        pltpu.sync_copy(dst3.at[sid, pl.ds(g * GCH, GCH)], dstv)

        fire(0, rows0, sem0)

        def body(i, _):
            j = 2 * i
            fire(j + 1, rows1, sem1)
            wait(rows0, sem0)
            scat(j, rows0)

            @pl.when(j + 2 < GCH)
            def _():
                fire(j + 2, rows0, sem0)
            wait(rows1, sem1)
            scat(j + 1, rows1)
            return ()
        lax.fori_loop(0, GCH // 2, body, (), unroll=False)

    plsc.subcore_barrier()

    def wbody(i, _):
        blk = sid + i * NSUB
        pltpu.sync_copy(acc.at[pl.ds(blk * ZROWS, ZROWS)],
                        out_hbm.at[pl.ds(blk * ZROWS, ZROWS)])
        return ()
    lax.fori_loop(0, nz, wbody, (), unroll=False)

    if cnt_parts is not None:
        onesv, zc, acc_cnt, cnt_out, cwb = cnt_parts

        def cwbody(i, _):
            blk = sid + i * NSUB
            pltpu.sync_copy(acc_cnt.at[pl.ds(blk * CNT_CB, CNT_CB)], cwb)
            pltpu.sync_copy(cwb, cnt_out.at[pl.ds(blk * CNT_CB, CNT_CB)])
            return ()
        lax.fori_loop(0, ncz, cwbody, (), unroll=False)


def _fill(ref, length, value):
    v = jnp.full((16,), value, jnp.float32)
    for j in range(0, length - 15, 16):
        ref[pl.ds(j, 16)] = v
    if length % 16:
        ref[pl.ds(length - 16, 16)] = v


def _make_segsum(with_cnt):
    out_type = [_f32((N, HALF)), _f32((N, HALF))]
    scratch = [
        pltpu.VMEM((GCH, CB), jnp.int32),
        pltpu.VMEM((GCH, CB), jnp.int32),
        pltpu.VMEM((CB, HALF), jnp.float32),
        pltpu.VMEM((CB, HALF), jnp.float32),
        pltpu.VMEM((ZROWS, HALF), jnp.float32),
        pltpu.VMEM_SHARED((N, HALF), jnp.float32),
        pltpu.SemaphoreType.DMA,
        pltpu.SemaphoreType.DMA,
    ]
    if with_cnt:
        out_type.append(_f32((N,)))
        scratch += [
            pltpu.VMEM((CB,), jnp.float32),
            pltpu.VMEM((CNT_CB,), jnp.float32),
            pltpu.VMEM_SHARED((N,), jnp.float32),
            pltpu.VMEM((CNT_CB,), jnp.float32),
        ]

    def body(ml, mr, src3, dst3, sl_out, sr_out, *rest):
        if with_cnt:
            cnt_out = rest[0]
            (srcv, dstv, rows0, rows1, zbuf, acc, sem0, sem1,
             onesv, zc, acc_cnt, cwb) = rest[1:]
        else:
            cnt_out = None
            srcv, dstv, rows0, rows1, zbuf, acc, sem0, sem1 = rest

        cid = lax.axis_index("c")
        sid = lax.axis_index("s")

        zero16 = jnp.zeros((16,), jnp.float32)

        def zfill(i, _):
            for j in range(HALF // 16):
                zbuf[i, pl.ds(j * 16, 16)] = zero16
            return ()
        lax.fori_loop(0, ZROWS, zfill, (), unroll=False)
        if with_cnt:
            _fill(onesv, CB, 1.0)
            _fill(zc, CNT_CB, 0.0)
            cnt_parts = (onesv, zc, acc_cnt, cnt_out, cwb)
        else:
            cnt_parts = None

        @pl.when(cid == 0)
        def _():
            _half_pipeline(sid, ml, src3, dst3, sl_out, srcv, dstv,
                           rows0, rows1, zbuf, acc, sem0, sem1, cnt_parts)

        @pl.when(cid == 1)
        def _():
            _half_pipeline(sid, mr, src3, dst3, sr_out, srcv, dstv,
                           rows0, rows1, zbuf, acc, sem0, sem1, None)

    return pl.kernel(body, out_type=out_type, mesh=_MESH,
                     scratch_types=scratch)


_segsum_cnt = _make_segsum(True)
_segsum = _make_segsum(False)


def _r1(b):
    return b.reshape(1, -1)


def _cast_w(ws):
    return [w.astype(jnp.bfloat16) if w.shape[0] > 1 else w for w in ws]


@jax.jit
def kernel(x, edge_index, edge_attr, params):
    src = edge_index[0].reshape(NSUB, NCHUNK, CB)
    dst = edge_index[1].reshape(NSUB, NCHUNK, CB)

    e_const = params["enc_edge"]["ln_b"][0]

    en = params["enc_node"]
    enc_w = []
    for l in en["mlp"]:
        enc_w += [l["w"], _r1(l["b"])]
    enc_w += [_r1(en["ln_g"]), _r1(en["ln_b"])]

    def msg_weights(t):
        lw = params["proc"][t]["edge_mlp"]
        w1 = lw[0]["w"]
        b1_eff = lw[0]["b"] + e_const * w1[D, :]
        return [w1[:D, :], _r1(b1_eff), lw[1]["w"], _r1(lw[1]["b"])]

    x0, ml, mr = _enc_call(x, _cast_w(enc_w + msg_weights(0)))

    sl, sr, cnt = _segsum_cnt(ml, mr, src, dst)
    recip = (1.0 / jnp.maximum(cnt, 1.0)).reshape(N, 1)

    for t in range(3):
        ls = params["proc"][t]["lin_self"]
        w = [ls["w"], _r1(ls["b"])] + msg_weights(t + 1)
        x0, ml, mr = _step_call(x0, sl, sr, recip, _cast_w(w))
        sl, sr = _segsum(ml, mr, src, dst)

    ls = params["proc"][3]["lin_self"]
    w = [ls["w"], _r1(ls["b"])]
    for l in params["dec"]:
        w += [l["w"], _r1(l["b"])]
    return _last_call(x0, sl, sr, recip, _cast_w(w))


# speedup vs baseline: 1.0278x; 1.0278x over previous
import functools

import jax
import jax.numpy as jnp
from jax import lax
from jax.experimental import pallas as pl
from jax.experimental.pallas import tpu as pltpu
from jax.experimental.pallas import tpu_sc as plsc

N = 10000
E = 160000
D = 256
HALF = 128
OUT_D = 3

NCORES = 2
NSUB = 16
EPT = E // NSUB
CB = 125
NCHUNK = EPT // CB
CNT_CB = 80
ZROWS = 40
NZB = N // ZROWS
NGRP = 5
GCH = NCHUNK // NGRP

BR = 1000
GRID = N // BR


def _bdot(x, w):
    return jnp.dot(x, w, preferred_element_type=jnp.float32)


def _msg(x, w1, b1, w2, b2):
    h = jnp.maximum(_bdot(x, w1) + b1, 0.0)
    return _bdot(h, w2) + b2


def _enc_body(x_ref, we1, be1, we2, be2, we3, be3, g_ref, b_ref,
              w1a, b1e, w2, b2, x0_ref, ml_ref, mr_ref):
    h = jnp.maximum(_bdot(x_ref[...], we1[...]) + be1[...], 0.0)
    h = jnp.maximum(_bdot(h, we2[...]) + be2[...], 0.0)
    h = _bdot(h, we3[...]) + be3[...]
    mu = jnp.mean(h, axis=1, keepdims=True)
    var = jnp.mean((h - mu) * (h - mu), axis=1, keepdims=True)
    x0 = (h - mu) / jnp.sqrt(var + 1e-5) * g_ref[...] + b_ref[...]
    x0_ref[...] = x0
    mm = _msg(x0, w1a[...], b1e[...], w2[...], b2[...])
    ml_ref[...] = mm[:, :HALF]
    mr_ref[...] = mm[:, HALF:]


def _step_body(x_ref, sl_ref, sr_ref, r_ref, ws, bs,
               w1a, b1e, w2, b2, xt_ref, ml_ref, mr_ref):
    aggr = jnp.concatenate([sl_ref[...], sr_ref[...]], axis=1) * r_ref[...]
    xt = _bdot(x_ref[...], ws[...]) + bs[...] + aggr
    xt_ref[...] = xt
    mm = _msg(xt, w1a[...], b1e[...], w2[...], b2[...])
    ml_ref[...] = mm[:, :HALF]
    mr_ref[...] = mm[:, HALF:]


def _last_body(x_ref, sl_ref, sr_ref, r_ref, ws, bs,
               wd1, bd1, wd2, bd2, wd3, bd3, o_ref):
    aggr = jnp.concatenate([sl_ref[...], sr_ref[...]], axis=1) * r_ref[...]
    xt = _bdot(x_ref[...], ws[...]) + bs[...] + aggr
    h = jnp.maximum(_bdot(xt, wd1[...]) + bd1[...], 0.0)
    h = jnp.maximum(_bdot(h, wd2[...]) + bd2[...], 0.0)
    o_ref[...] = _bdot(h, wd3[...]) + bd3[...]


def _row_spec(width):
    return pl.BlockSpec((BR, width), lambda i: (i, 0))


def _full_spec(shape):
    return pl.BlockSpec(shape, lambda i: tuple(0 for _ in shape))


def _wspec(a):
    return _full_spec(a.shape)


def _f32(shape):
    return jax.ShapeDtypeStruct(shape, jnp.float32)


def _enc_call(x, weights):
    in_specs = [_row_spec(D)] + [_wspec(w) for w in weights]
    return pl.pallas_call(
        _enc_body,
        grid=(GRID,),
        in_specs=in_specs,
        out_specs=[_row_spec(D), _row_spec(HALF), _row_spec(HALF)],
        out_shape=[_f32((N, D)), _f32((N, HALF)), _f32((N, HALF))],
    )(x, *weights)


def _step_call(x, sl, sr, recip, weights):
    in_specs = [_row_spec(D), _row_spec(HALF), _row_spec(HALF), _row_spec(1)]
    in_specs += [_wspec(w) for w in weights]
    return pl.pallas_call(
        _step_body,
        grid=(GRID,),
        in_specs=in_specs,
        out_specs=[_row_spec(D), _row_spec(HALF), _row_spec(HALF)],
        out_shape=[_f32((N, D)), _f32((N, HALF)), _f32((N, HALF))],
    )(x, sl, sr, recip, *weights)


def _last_call(x, sl, sr, recip, weights):
    in_specs = [_row_spec(D), _row_spec(HALF), _row_spec(HALF), _row_spec(1)]
    in_specs += [_wspec(w) for w in weights]
    return pl.pallas_call(
        _last_body,
        grid=(GRID,),
        in_specs=in_specs,
        out_specs=[_row_spec(OUT_D)],
        out_shape=[_f32((N, OUT_D))],
    )(x, sl, sr, recip, *weights)[0]


_MESH = plsc.VectorSubcoreMesh(
    core_axis_name="c", subcore_axis_name="s",
    num_cores=NCORES, num_subcores=NSUB)

_CNT_BLK = N // CNT_CB


def _half_pipeline(sid, m_hbm, src3, dst3, out_hbm, srcv0, dstv0, srcv1,
                   dstv1, rows0, rows1, zbuf, acc, sem0, sem1, zsem,
                   isem0, isem1, cnt_parts):
    idx = [(srcv0, dstv0, isem0), (srcv1, dstv1, isem1)]

    def fire_idx(g, b):
        sv, dv, isem = idx[b]
        pltpu.async_copy(src3.at[sid, pl.ds(g * GCH, GCH)], sv, isem)
        pltpu.async_copy(dst3.at[sid, pl.ds(g * GCH, GCH)], dv, isem)

    def drain_idx(b):
        sv, dv, isem = idx[b]
        pltpu.make_async_copy(src3.at[sid, pl.ds(0, GCH)], sv, isem).wait()
        pltpu.make_async_copy(dst3.at[sid, pl.ds(0, GCH)], dv, isem).wait()

    fire_idx(0, 0)

    nz = jnp.where(sid < NZB % NSUB, NZB // NSUB + 1, NZB // NSUB)

    def zbody(i, _):
        blk = sid + i * NSUB
        pltpu.async_copy(zbuf, acc.at[pl.ds(blk * ZROWS, ZROWS)], zsem)
        return ()
    lax.fori_loop(0, nz, zbody, (), unroll=False)

    if cnt_parts is not None:
        onesv, zc, acc_cnt, cnt_out, cwb = cnt_parts
        ncz = jnp.where(sid < _CNT_BLK % NSUB, _CNT_BLK // NSUB + 1,
                        _CNT_BLK // NSUB)

        def czbody(i, _):
            blk = sid + i * NSUB
            pltpu.sync_copy(zc, acc_cnt.at[pl.ds(blk * CNT_CB, CNT_CB)])
            return ()
        lax.fori_loop(0, ncz, czbody, (), unroll=False)

    def fire(srcv, j, buf, sem):
        pltpu.async_copy(m_hbm.at[srcv.at[j]], buf, sem)

    def wait(buf, sem):
        pltpu.make_async_copy(m_hbm.at[srcv0.at[0]], buf, sem).wait()

    def scat(dstv, j, buf):
        pltpu.sync_copy(buf, acc.at[dstv.at[j]], add=True)
        if cnt_parts is not None:
            pltpu.sync_copy(cnt_parts[0], cnt_parts[2].at[dstv.at[j]],
                            add=True)

    drain_idx(0)
    fire(srcv0, 0, rows0, sem0)

    def zdrain(i, _):
        pltpu.make_async_copy(zbuf, acc.at[pl.ds(0, ZROWS)], zsem).wait()
        return ()
    lax.fori_loop(0, nz, zdrain, (), unroll=False)
    plsc.subcore_barrier()

    for g in range(NGRP):
        sv, dv, _ = idx[g % 2]
        if g > 0:
            drain_idx(g % 2)
            fire(sv, 0, rows0, sem0)
        if g + 1 < NGRP:
            fire_idx(g + 1, (g + 1) % 2)

        def body(i, _):
            j = 2 * i
            fire(sv, j + 1, rows1, sem1)
            wait(rows0, sem0)
            scat(dv, j, rows0)

            @pl.when(j + 2 < GCH)
            def _():
                fire(sv, j + 2, rows0, sem0)
            wait(rows1, sem1)
            scat(dv, j + 1, rows1)
            return ()
        lax.fori_loop(0, GCH // 2, body, (), unroll=False)

    plsc.subcore_barrier()

    def wbody(i, _):
        blk = sid + i * NSUB
        pltpu.async_copy(acc.at[pl.ds(blk * ZROWS, ZROWS)],
                         out_hbm.at[pl.ds(blk * ZROWS, ZROWS)], zsem)
        return ()
    lax.fori_loop(0, nz, wbody, (), unroll=False)

    def wdrain(i, _):
        pltpu.make_async_copy(acc.at[pl.ds(0, ZROWS)],
                              out_hbm.at[pl.ds(0, ZROWS)], zsem).wait()
        return ()
    lax.fori_loop(0, nz, wdrain, (), unroll=False)

    if cnt_parts is not None:
        onesv, zc, acc_cnt, cnt_out, cwb = cnt_parts

        def cwbody(i, _):
            blk = sid + i * NSUB
            pltpu.sync_copy(acc_cnt.at[pl.ds(blk * CNT_CB, CNT_CB)], cwb)
            pltpu.sync_copy(cwb, cnt_out.at[pl.ds(blk * CNT_CB, CNT_CB)])
            return ()
        lax.fori_loop(0, ncz, cwbody, (), unroll=False)


def _fill(ref, length, value):
    v = jnp.full((16,), value, jnp.float32)
    for j in range(0, length - 15, 16):
        ref[pl.ds(j, 16)] = v
    if length % 16:
        ref[pl.ds(length - 16, 16)] = v


def _make_segsum(with_cnt):
    out_type = [_f32((N, HALF)), _f32((N, HALF))]
    scratch = [
        pltpu.VMEM((GCH, CB), jnp.int32),
        pltpu.VMEM((GCH, CB), jnp.int32),
        pltpu.VMEM((GCH, CB), jnp.int32),
        pltpu.VMEM((GCH, CB), jnp.int32),
        pltpu.VMEM((CB, HALF), jnp.float32),
        pltpu.VMEM((CB, HALF), jnp.float32),
        pltpu.VMEM((ZROWS, HALF), jnp.float32),
        pltpu.VMEM_SHARED((N, HALF), jnp.float32),
        pltpu.SemaphoreType.DMA,
        pltpu.SemaphoreType.DMA,
        pltpu.SemaphoreType.DMA,
        pltpu.SemaphoreType.DMA,
        pltpu.SemaphoreType.DMA,
    ]
    if with_cnt:
        out_type.append(_f32((N,)))
        scratch += [
            pltpu.VMEM((CB,), jnp.float32),
            pltpu.VMEM((CNT_CB,), jnp.float32),
            pltpu.VMEM_SHARED((N,), jnp.float32),
            pltpu.VMEM((CNT_CB,), jnp.float32),
        ]

    def body(ml, mr, src3, dst3, sl_out, sr_out, *rest):
        if with_cnt:
            cnt_out = rest[0]
            (srcv0, dstv0, srcv1, dstv1, rows0, rows1, zbuf, acc,
             sem0, sem1, zsem, isem0, isem1,
             onesv, zc, acc_cnt, cwb) = rest[1:]
        else:
            cnt_out = None
            (srcv0, dstv0, srcv1, dstv1, rows0, rows1, zbuf, acc,
             sem0, sem1, zsem, isem0, isem1) = rest

        cid = lax.axis_index("c")
        sid = lax.axis_index("s")

        zero16 = jnp.zeros((16,), jnp.float32)

        def zfill(i, _):
            for j in range(HALF // 16):
                zbuf[i, pl.ds(j * 16, 16)] = zero16
            return ()
        lax.fori_loop(0, ZROWS, zfill, (), unroll=False)
        if with_cnt:
            _fill(onesv, CB, 1.0)
            _fill(zc, CNT_CB, 0.0)
            cnt_parts = (onesv, zc, acc_cnt, cnt_out, cwb)
        else:
            cnt_parts = None

        @pl.when(cid == 0)
        def _():
            _half_pipeline(sid, ml, src3, dst3, sl_out, srcv0, dstv0,
                           srcv1, dstv1, rows0, rows1, zbuf, acc,
                           sem0, sem1, zsem, isem0, isem1, cnt_parts)

        @pl.when(cid == 1)
        def _():
            _half_pipeline(sid, mr, src3, dst3, sr_out, srcv0, dstv0,
                           srcv1, dstv1, rows0, rows1, zbuf, acc,
                           sem0, sem1, zsem, isem0, isem1, None)

    return pl.kernel(body, out_type=out_type, mesh=_MESH,
                     scratch_types=scratch)


_segsum_cnt = _make_segsum(True)
_segsum = _make_segsum(False)


def _r1(b):
    return b.reshape(1, -1)


def _cast_w(ws):
    return ws


@jax.jit
def kernel(x, edge_index, edge_attr, params):
    src = edge_index[0].reshape(NSUB, NCHUNK, CB)
    dst = edge_index[1].reshape(NSUB, NCHUNK, CB)

    e_const = params["enc_edge"]["ln_b"][0]

    en = params["enc_node"]
    enc_w = []
    for l in en["mlp"]:
        enc_w += [l["w"], _r1(l["b"])]
    enc_w += [_r1(en["ln_g"]), _r1(en["ln_b"])]

    def msg_weights(t):
        lw = params["proc"][t]["edge_mlp"]
        w1 = lw[0]["w"]
        b1_eff = lw[0]["b"] + e_const * w1[D, :]
        return [w1[:D, :], _r1(b1_eff), lw[1]["w"], _r1(lw[1]["b"])]

    x0, ml, mr = _enc_call(x, _cast_w(enc_w + msg_weights(0)))

    sl, sr, cnt = _segsum_cnt(ml, mr, src, dst)
    recip = (1.0 / jnp.maximum(cnt, 1.0)).reshape(N, 1)

    for t in range(3):
        ls = params["proc"][t]["lin_self"]
        w = [ls["w"], _r1(ls["b"])] + msg_weights(t + 1)
        x0, ml, mr = _step_call(x0, sl, sr, recip, _cast_w(w))
        sl, sr = _segsum(ml, mr, src, dst)

    ls = params["proc"][3]["lin_self"]
    w = [ls["w"], _r1(ls["b"])]
    for l in params["dec"]:
        w += [l["w"], _r1(l["b"])]
    return _last_call(x0, sl, sr, recip, _cast_w(w))


# speedup vs baseline: 1.2017x; 1.1692x over previous
import functools

import jax
import jax.numpy as jnp
from jax import lax
from jax.experimental import pallas as pl
from jax.experimental.pallas import tpu as pltpu
from jax.experimental.pallas import tpu_sc as plsc

N = 10000
E = 160000
D = 256
HALF = 128
OUT_D = 3

NCORES = 2
NSUB = 16
EPT = E // NSUB
CB = 40
NCHUNK = EPT // CB
RING = 5
CNT_CB = 80
ZROWS = 16
NZB = N // ZROWS

BR = 1000
GRID = N // BR


def _bdot(x, w):
    return jnp.dot(x, w, preferred_element_type=jnp.float32)


def _msg(x, w1, b1, w2, b2):
    h = jnp.maximum(_bdot(x, w1) + b1, 0.0)
    return _bdot(h, w2) + b2


def _enc_body(x_ref, we1, be1, we2, be2, we3, be3, g_ref, b_ref,
              w1a, b1e, w2, b2, x0_ref, ml_ref, mr_ref):
    h = jnp.maximum(_bdot(x_ref[...], we1[...]) + be1[...], 0.0)
    h = jnp.maximum(_bdot(h, we2[...]) + be2[...], 0.0)
    h = _bdot(h, we3[...]) + be3[...]
    mu = jnp.mean(h, axis=1, keepdims=True)
    var = jnp.mean((h - mu) * (h - mu), axis=1, keepdims=True)
    x0 = (h - mu) / jnp.sqrt(var + 1e-5) * g_ref[...] + b_ref[...]
    x0_ref[...] = x0
    mm = _msg(x0, w1a[...], b1e[...], w2[...], b2[...])
    ml_ref[...] = mm[:, :HALF]
    mr_ref[...] = mm[:, HALF:]


def _step_body(x_ref, sl_ref, sr_ref, r_ref, ws, bs,
               w1a, b1e, w2, b2, xt_ref, ml_ref, mr_ref):
    aggr = jnp.concatenate([sl_ref[...], sr_ref[...]], axis=1) * r_ref[...]
    xt = _bdot(x_ref[...], ws[...]) + bs[...] + aggr
    xt_ref[...] = xt
    mm = _msg(xt, w1a[...], b1e[...], w2[...], b2[...])
    ml_ref[...] = mm[:, :HALF]
    mr_ref[...] = mm[:, HALF:]


def _last_body(x_ref, sl_ref, sr_ref, r_ref, ws, bs,
               wd1, bd1, wd2, bd2, wd3, bd3, o_ref):
    aggr = jnp.concatenate([sl_ref[...], sr_ref[...]], axis=1) * r_ref[...]
    xt = _bdot(x_ref[...], ws[...]) + bs[...] + aggr
    h = jnp.maximum(_bdot(xt, wd1[...]) + bd1[...], 0.0)
    h = jnp.maximum(_bdot(h, wd2[...]) + bd2[...], 0.0)
    o_ref[...] = _bdot(h, wd3[...]) + bd3[...]


def _row_spec(width):
    return pl.BlockSpec((BR, width), lambda i: (i, 0))


def _full_spec(shape):
    return pl.BlockSpec(shape, lambda i: tuple(0 for _ in shape))


def _wspec(a):
    return _full_spec(a.shape)


def _f32(shape):
    return jax.ShapeDtypeStruct(shape, jnp.float32)


def _enc_call(x, weights):
    in_specs = [_row_spec(D)] + [_wspec(w) for w in weights]
    return pl.pallas_call(
        _enc_body,
        grid=(GRID,),
        in_specs=in_specs,
        out_specs=[_row_spec(D), _row_spec(HALF), _row_spec(HALF)],
        out_shape=[_f32((N, D)), _f32((N, HALF)), _f32((N, HALF))],
    )(x, *weights)


def _step_call(x, sl, sr, recip, weights):
    in_specs = [_row_spec(D), _row_spec(HALF), _row_spec(HALF), _row_spec(1)]
    in_specs += [_wspec(w) for w in weights]
    return pl.pallas_call(
        _step_body,
        grid=(GRID,),
        in_specs=in_specs,
        out_specs=[_row_spec(D), _row_spec(HALF), _row_spec(HALF)],
        out_shape=[_f32((N, D)), _f32((N, HALF)), _f32((N, HALF))],
    )(x, sl, sr, recip, *weights)


def _last_call(x, sl, sr, recip, weights):
    in_specs = [_row_spec(D), _row_spec(HALF), _row_spec(HALF), _row_spec(1)]
    in_specs += [_wspec(w) for w in weights]
    return pl.pallas_call(
        _last_body,
        grid=(GRID,),
        in_specs=in_specs,
        out_specs=[_row_spec(OUT_D)],
        out_shape=[_f32((N, OUT_D))],
    )(x, sl, sr, recip, *weights)[0]


_MESH = plsc.VectorSubcoreMesh(
    core_axis_name="c", subcore_axis_name="s",
    num_cores=NCORES, num_subcores=NSUB)

_CNT_BLK = N // CNT_CB


def _half_pipeline(sid, m_hbm, src1, dst3, out_hbm, srcv, dstv,
                   rows, sems, zbuf, acc, zsem, isem, cnt_parts):
    pltpu.async_copy(src1.at[pl.ds(sid * EPT, EPT)], srcv, isem)

    nz = jnp.where(sid < NZB % NSUB, NZB // NSUB + 1, NZB // NSUB)

    def zbody(i, _):
        blk = sid + i * NSUB
        pltpu.async_copy(zbuf, acc.at[pl.ds(blk * ZROWS, ZROWS)], zsem)
        return ()
    lax.fori_loop(0, nz, zbody, (), unroll=False)

    if cnt_parts is not None:
        onesv, zc, acc_cnt, cnt_out, cwb = cnt_parts
        ncz = jnp.where(sid < _CNT_BLK % NSUB, _CNT_BLK // NSUB + 1,
                        _CNT_BLK // NSUB)

        def czbody(i, _):
            blk = sid + i * NSUB
            pltpu.sync_copy(zc, acc_cnt.at[pl.ds(blk * CNT_CB, CNT_CB)])
            return ()
        lax.fori_loop(0, ncz, czbody, (), unroll=False)

    def fire(j, b):
        pltpu.async_copy(dst3.at[sid, j], dstv.at[b], sems[b])
        pltpu.async_copy(m_hbm.at[srcv.at[pl.ds(j * CB, CB)]], rows[b],
                         sems[b])

    def wait(b):
        pltpu.make_async_copy(dst3.at[sid, 0], dstv.at[b], sems[b]).wait()
        pltpu.make_async_copy(m_hbm.at[srcv.at[pl.ds(0, CB)]], rows[b],
                              sems[b]).wait()

    def scat(j, b):
        pltpu.sync_copy(rows[b], acc.at[dstv.at[b]], add=True)
        if cnt_parts is not None:
            pltpu.sync_copy(cnt_parts[0], cnt_parts[2].at[dstv.at[b]],
                            add=True)

    pltpu.make_async_copy(src1.at[pl.ds(0, EPT)], srcv, isem).wait()
    for k in range(RING - 1):
        fire(k, k)

    def zdrain(i, _):
        pltpu.make_async_copy(zbuf, acc.at[pl.ds(0, ZROWS)], zsem).wait()
        return ()
    lax.fori_loop(0, nz, zdrain, (), unroll=False)
    plsc.subcore_barrier()

    def body(i, _):
        j = RING * i
        fire(j + RING - 1, RING - 1)
        for b in range(RING - 1):
            wait(b)
            scat(j + b, b)

            @pl.when(j + b + RING < NCHUNK)
            def _():
                fire(j + b + RING, b)
        wait(RING - 1)
        scat(j + RING - 1, RING - 1)
        return ()
    lax.fori_loop(0, NCHUNK // RING, body, (), unroll=False)

    plsc.subcore_barrier()

    def wbody(i, _):
        blk = sid + i * NSUB
        pltpu.async_copy(acc.at[pl.ds(blk * ZROWS, ZROWS)],
                         out_hbm.at[pl.ds(blk * ZROWS, ZROWS)], zsem)
        return ()
    lax.fori_loop(0, nz, wbody, (), unroll=False)

    def wdrain(i, _):
        pltpu.make_async_copy(acc.at[pl.ds(0, ZROWS)],
                              out_hbm.at[pl.ds(0, ZROWS)], zsem).wait()
        return ()
    lax.fori_loop(0, nz, wdrain, (), unroll=False)

    if cnt_parts is not None:
        onesv, zc, acc_cnt, cnt_out, cwb = cnt_parts

        def cwbody(i, _):
            blk = sid + i * NSUB
            pltpu.sync_copy(acc_cnt.at[pl.ds(blk * CNT_CB, CNT_CB)], cwb)
            pltpu.sync_copy(cwb, cnt_out.at[pl.ds(blk * CNT_CB, CNT_CB)])
            return ()
        lax.fori_loop(0, ncz, cwbody, (), unroll=False)


def _fill(ref, length, value):
    v = jnp.full((16,), value, jnp.float32)
    for j in range(0, length - 15, 16):
        ref[pl.ds(j, 16)] = v
    if length % 16:
        ref[pl.ds(length - 16, 16)] = v


def _make_segsum(with_cnt):
    out_type = [_f32((N, HALF)), _f32((N, HALF))]
    scratch = [
        pltpu.VMEM((EPT,), jnp.int32),
        pltpu.VMEM((RING, CB), jnp.int32),
    ]
    scratch += [pltpu.VMEM((CB, HALF), jnp.float32)
                for _ in range(RING)]
    scratch += [
        pltpu.VMEM((ZROWS, HALF), jnp.float32),
        pltpu.VMEM_SHARED((N, HALF), jnp.float32),
    ]
    scratch += [pltpu.SemaphoreType.DMA for _ in range(RING)]
    scratch += [
        pltpu.SemaphoreType.DMA,
        pltpu.SemaphoreType.DMA,
    ]
    if with_cnt:
        out_type.append(_f32((N,)))
        scratch += [
            pltpu.VMEM((CB,), jnp.float32),
            pltpu.VMEM((CNT_CB,), jnp.float32),
            pltpu.VMEM_SHARED((N,), jnp.float32),
            pltpu.VMEM((CNT_CB,), jnp.float32),
        ]

    def body(ml, mr, src3, dst3, sl_out, sr_out, *rest):
        if with_cnt:
            cnt_out = rest[0]
            rest = rest[1:]
        else:
            cnt_out = None
        srcv, dstv = rest[0], rest[1]
        rows = list(rest[2:2 + RING])
        zbuf, acc = rest[2 + RING], rest[3 + RING]
        sems = list(rest[4 + RING:4 + 2 * RING])
        zsem, isem = rest[4 + 2 * RING], rest[5 + 2 * RING]
        if with_cnt:
            onesv, zc, acc_cnt, cwb = rest[6 + 2 * RING:]

        cid = lax.axis_index("c")
        sid = lax.axis_index("s")

        zero16 = jnp.zeros((16,), jnp.float32)

        def zfill(i, _):
            for j in range(HALF // 16):
                zbuf[i, pl.ds(j * 16, 16)] = zero16
            return ()
        lax.fori_loop(0, ZROWS, zfill, (), unroll=False)
        if with_cnt:
            _fill(onesv, CB, 1.0)
            _fill(zc, CNT_CB, 0.0)
            cnt_parts = (onesv, zc, acc_cnt, cnt_out, cwb)
        else:
            cnt_parts = None

        @pl.when(cid == 0)
        def _():
            _half_pipeline(sid, ml, src3, dst3, sl_out, srcv, dstv,
                           rows, sems, zbuf, acc, zsem, isem, cnt_parts)

        @pl.when(cid == 1)
        def _():
            _half_pipeline(sid, mr, src3, dst3, sr_out, srcv, dstv,
                           rows, sems, zbuf, acc, zsem, isem, None)

    return pl.kernel(body, out_type=out_type, mesh=_MESH,
                     scratch_types=scratch)


_segsum_cnt = _make_segsum(True)
_segsum = _make_segsum(False)


def _r1(b):
    return b.reshape(1, -1)


def _cast_w(ws):
    return ws


@jax.jit
def kernel(x, edge_index, edge_attr, params):
    src = edge_index[0]
    dst = edge_index[1].reshape(NSUB, NCHUNK, CB)

    e_const = params["enc_edge"]["ln_b"][0]

    en = params["enc_node"]
    enc_w = []
    for l in en["mlp"]:
        enc_w += [l["w"], _r1(l["b"])]
    enc_w += [_r1(en["ln_g"]), _r1(en["ln_b"])]

    def msg_weights(t):
        lw = params["proc"][t]["edge_mlp"]
        w1 = lw[0]["w"]
        b1_eff = lw[0]["b"] + e_const * w1[D, :]
        return [w1[:D, :], _r1(b1_eff), lw[1]["w"], _r1(lw[1]["b"])]

    x0, ml, mr = _enc_call(x, _cast_w(enc_w + msg_weights(0)))

    sl, sr, cnt = _segsum_cnt(ml, mr, src, dst)
    recip = (1.0 / jnp.maximum(cnt, 1.0)).reshape(N, 1)

    for t in range(3):
        ls = params["proc"][t]["lin_self"]
        w = [ls["w"], _r1(ls["b"])] + msg_weights(t + 1)
        x0, ml, mr = _step_call(x0, sl, sr, recip, _cast_w(w))
        sl, sr = _segsum(ml, mr, src, dst)

    ls = params["proc"][3]["lin_self"]
    w = [ls["w"], _r1(ls["b"])]
    for l in params["dec"]:
        w += [l["w"], _r1(l["b"])]
    return _last_call(x0, sl, sr, recip, _cast_w(w))


# speedup vs baseline: 1.2420x; 1.0335x over previous
import functools

import jax
import jax.numpy as jnp
from jax import lax
from jax.experimental import pallas as pl
from jax.experimental.pallas import tpu as pltpu
from jax.experimental.pallas import tpu_sc as plsc

N = 10000
E = 160000
D = 256
HALF = 128
OUT_D = 3

NCORES = 2
NSUB = 16
EPT = E // NSUB
CB = 40
NCHUNK = EPT // CB
RING = 5
CNT_CB = 80
ZROWS = 16
NZB = N // ZROWS

BR = 2000
GRID = N // BR


def _bdot(x, w):
    return jnp.dot(x, w, preferred_element_type=jnp.float32)


def _msg(x, w1, b1, w2, b2):
    h = jnp.maximum(_bdot(x, w1) + b1, 0.0)
    return _bdot(h, w2) + b2


def _enc_body(x_ref, we1, be1, we2, be2, we3, be3, g_ref, b_ref,
              w1a, b1e, w2, b2, x0_ref, ml_ref, mr_ref):
    h = jnp.maximum(_bdot(x_ref[...], we1[...]) + be1[...], 0.0)
    h = jnp.maximum(_bdot(h, we2[...]) + be2[...], 0.0)
    h = _bdot(h, we3[...]) + be3[...]
    mu = jnp.mean(h, axis=1, keepdims=True)
    var = jnp.mean((h - mu) * (h - mu), axis=1, keepdims=True)
    x0 = (h - mu) / jnp.sqrt(var + 1e-5) * g_ref[...] + b_ref[...]
    x0_ref[...] = x0
    mm = _msg(x0, w1a[...], b1e[...], w2[...], b2[...])
    ml_ref[...] = mm[:, :HALF]
    mr_ref[...] = mm[:, HALF:]


def _step_body(x_ref, sl_ref, sr_ref, r_ref, ws, bs,
               w1a, b1e, w2, b2, xt_ref, ml_ref, mr_ref):
    aggr = jnp.concatenate([sl_ref[...], sr_ref[...]], axis=1) * r_ref[...]
    xt = _bdot(x_ref[...], ws[...]) + bs[...] + aggr
    xt_ref[...] = xt
    mm = _msg(xt, w1a[...], b1e[...], w2[...], b2[...])
    ml_ref[...] = mm[:, :HALF]
    mr_ref[...] = mm[:, HALF:]


def _last_body(x_ref, sl_ref, sr_ref, r_ref, ws, bs,
               wd1, bd1, wd2, bd2, wd3, bd3, o_ref):
    aggr = jnp.concatenate([sl_ref[...], sr_ref[...]], axis=1) * r_ref[...]
    xt = _bdot(x_ref[...], ws[...]) + bs[...] + aggr
    h = jnp.maximum(_bdot(xt, wd1[...]) + bd1[...], 0.0)
    h = jnp.maximum(_bdot(h, wd2[...]) + bd2[...], 0.0)
    o_ref[...] = _bdot(h, wd3[...]) + bd3[...]


def _row_spec(width):
    return pl.BlockSpec((BR, width), lambda i: (i, 0))


def _full_spec(shape):
    return pl.BlockSpec(shape, lambda i: tuple(0 for _ in shape))


def _wspec(a):
    return _full_spec(a.shape)


def _f32(shape):
    return jax.ShapeDtypeStruct(shape, jnp.float32)


def _enc_call(x, weights):
    in_specs = [_row_spec(D)] + [_wspec(w) for w in weights]
    return pl.pallas_call(
        _enc_body,
        grid=(GRID,),
        in_specs=in_specs,
        out_specs=[_row_spec(D), _row_spec(HALF), _row_spec(HALF)],
        out_shape=[_f32((N, D)), _f32((N, HALF)), _f32((N, HALF))],
    )(x, *weights)


def _step_call(x, sl, sr, recip, weights):
    in_specs = [_row_spec(D), _row_spec(HALF), _row_spec(HALF), _row_spec(1)]
    in_specs += [_wspec(w) for w in weights]
    return pl.pallas_call(
        _step_body,
        grid=(GRID,),
        in_specs=in_specs,
        out_specs=[_row_spec(D), _row_spec(HALF), _row_spec(HALF)],
        out_shape=[_f32((N, D)), _f32((N, HALF)), _f32((N, HALF))],
    )(x, sl, sr, recip, *weights)


def _last_call(x, sl, sr, recip, weights):
    in_specs = [_row_spec(D), _row_spec(HALF), _row_spec(HALF), _row_spec(1)]
    in_specs += [_wspec(w) for w in weights]
    return pl.pallas_call(
        _last_body,
        grid=(GRID,),
        in_specs=in_specs,
        out_specs=[_row_spec(OUT_D)],
        out_shape=[_f32((N, OUT_D))],
    )(x, sl, sr, recip, *weights)[0]


_MESH = plsc.VectorSubcoreMesh(
    core_axis_name="c", subcore_axis_name="s",
    num_cores=NCORES, num_subcores=NSUB)

_CNT_BLK = N // CNT_CB


def _half_pipeline(sid, m_hbm, src1, dst3, out_hbm, srcv, dstv,
                   rows, sems, ssems, zbuf, acc, zsem, isem, cnt_parts):
    pltpu.async_copy(src1.at[pl.ds(sid * EPT, EPT)], srcv, isem)

    nz = jnp.where(sid < NZB % NSUB, NZB // NSUB + 1, NZB // NSUB)

    def zbody(i, _):
        blk = sid + i * NSUB
        pltpu.async_copy(zbuf, acc.at[pl.ds(blk * ZROWS, ZROWS)], zsem)
        return ()
    lax.fori_loop(0, nz, zbody, (), unroll=False)

    if cnt_parts is not None:
        onesv, zc, acc_cnt, cnt_out, cwb = cnt_parts
        ncz = jnp.where(sid < _CNT_BLK % NSUB, _CNT_BLK // NSUB + 1,
                        _CNT_BLK // NSUB)

        def czbody(i, _):
            blk = sid + i * NSUB
            pltpu.sync_copy(zc, acc_cnt.at[pl.ds(blk * CNT_CB, CNT_CB)])
            return ()
        lax.fori_loop(0, ncz, czbody, (), unroll=False)

    def fire(j, b):
        pltpu.async_copy(dst3.at[sid, j], dstv.at[b], sems[b])
        pltpu.async_copy(m_hbm.at[srcv.at[pl.ds(j * CB, CB)]], rows[b],
                         sems[b])

    def wait(b):
        pltpu.make_async_copy(dst3.at[sid, 0], dstv.at[b], sems[b]).wait()
        pltpu.make_async_copy(m_hbm.at[srcv.at[pl.ds(0, CB)]], rows[b],
                              sems[b]).wait()

    def scat(j, b):
        pltpu.async_copy(rows[b], acc.at[dstv.at[b]], ssems[b], add=True)
        if cnt_parts is not None:
            pltpu.async_copy(cnt_parts[0], cnt_parts[2].at[dstv.at[b]],
                             ssems[b], add=True)

    def wait_scat(b):
        pltpu.make_async_copy(rows[b], acc.at[dstv.at[b]], ssems[b]).wait()
        if cnt_parts is not None:
            pltpu.make_async_copy(cnt_parts[0], cnt_parts[2].at[dstv.at[b]],
                                  ssems[b]).wait()

    pltpu.make_async_copy(src1.at[pl.ds(0, EPT)], srcv, isem).wait()
    for k in range(RING):
        fire(k, k)

    def zdrain(i, _):
        pltpu.make_async_copy(zbuf, acc.at[pl.ds(0, ZROWS)], zsem).wait()
        return ()
    lax.fori_loop(0, nz, zdrain, (), unroll=False)
    plsc.subcore_barrier()

    def body(i, _):
        j = RING * i
        for b in range(RING):
            wait(b)
            scat(j + b, b)

            @pl.when(j + b + RING < NCHUNK)
            def _():
                wait_scat(b)
                fire(j + b + RING, b)
        return ()
    lax.fori_loop(0, NCHUNK // RING, body, (), unroll=False)

    for b in range(RING):
        wait_scat(b)

    plsc.subcore_barrier()

    def wbody(i, _):
        blk = sid + i * NSUB
        pltpu.async_copy(acc.at[pl.ds(blk * ZROWS, ZROWS)],
                         out_hbm.at[pl.ds(blk * ZROWS, ZROWS)], zsem)
        return ()
    lax.fori_loop(0, nz, wbody, (), unroll=False)

    def wdrain(i, _):
        pltpu.make_async_copy(acc.at[pl.ds(0, ZROWS)],
                              out_hbm.at[pl.ds(0, ZROWS)], zsem).wait()
        return ()
    lax.fori_loop(0, nz, wdrain, (), unroll=False)

    if cnt_parts is not None:
        onesv, zc, acc_cnt, cnt_out, cwb = cnt_parts

        def cwbody(i, _):
            blk = sid + i * NSUB
            pltpu.sync_copy(acc_cnt.at[pl.ds(blk * CNT_CB, CNT_CB)], cwb)
            pltpu.sync_copy(cwb, cnt_out.at[pl.ds(blk * CNT_CB, CNT_CB)])
            return ()
        lax.fori_loop(0, ncz, cwbody, (), unroll=False)


def _fill(ref, length, value):
    v = jnp.full((16,), value, jnp.float32)
    for j in range(0, length - 15, 16):
        ref[pl.ds(j, 16)] = v
    if length % 16:
        ref[pl.ds(length - 16, 16)] = v


def _make_segsum(with_cnt):
    out_type = [_f32((N, HALF)), _f32((N, HALF))]
    scratch = [
        pltpu.VMEM((EPT,), jnp.int32),
        pltpu.VMEM((RING, CB), jnp.int32),
    ]
    scratch += [pltpu.VMEM((CB, HALF), jnp.float32)
                for _ in range(RING)]
    scratch += [
        pltpu.VMEM((ZROWS, HALF), jnp.float32),
        pltpu.VMEM_SHARED((N, HALF), jnp.float32),
    ]
    scratch += [pltpu.SemaphoreType.DMA for _ in range(RING)]
    scratch += [pltpu.SemaphoreType.DMA for _ in range(RING)]
    scratch += [
        pltpu.SemaphoreType.DMA,
        pltpu.SemaphoreType.DMA,
    ]
    if with_cnt:
        out_type.append(_f32((N,)))
        scratch += [
            pltpu.VMEM((CB,), jnp.float32),
            pltpu.VMEM((CNT_CB,), jnp.float32),
            pltpu.VMEM_SHARED((N,), jnp.float32),
            pltpu.VMEM((CNT_CB,), jnp.float32),
        ]

    def body(ml, mr, src3, dst3, sl_out, sr_out, *rest):
        if with_cnt:
            cnt_out = rest[0]
            rest = rest[1:]
        else:
            cnt_out = None
        srcv, dstv = rest[0], rest[1]
        rows = list(rest[2:2 + RING])
        zbuf, acc = rest[2 + RING], rest[3 + RING]
        sems = list(rest[4 + RING:4 + 2 * RING])
        ssems = list(rest[4 + 2 * RING:4 + 3 * RING])
        zsem, isem = rest[4 + 3 * RING], rest[5 + 3 * RING]
        if with_cnt:
            onesv, zc, acc_cnt, cwb = rest[6 + 3 * RING:]

        cid = lax.axis_index("c")
        sid = lax.axis_index("s")

        zero16 = jnp.zeros((16,), jnp.float32)

        def zfill(i, _):
            for j in range(HALF // 16):
                zbuf[i, pl.ds(j * 16, 16)] = zero16
            return ()
        lax.fori_loop(0, ZROWS, zfill, (), unroll=False)
        if with_cnt:
            _fill(onesv, CB, 1.0)
            _fill(zc, CNT_CB, 0.0)
            cnt_parts = (onesv, zc, acc_cnt, cnt_out, cwb)
        else:
            cnt_parts = None

        @pl.when(cid == 0)
        def _():
            _half_pipeline(sid, ml, src3, dst3, sl_out, srcv, dstv,
                           rows, sems, ssems, zbuf, acc, zsem, isem,
                           cnt_parts)

        @pl.when(cid == 1)
        def _():
            _half_pipeline(sid, mr, src3, dst3, sr_out, srcv, dstv,
                           rows, sems, ssems, zbuf, acc, zsem, isem, None)

    return pl.kernel(body, out_type=out_type, mesh=_MESH,
                     scratch_types=scratch)


_segsum_cnt = _make_segsum(True)
_segsum = _make_segsum(False)


def _r1(b):
    return b.reshape(1, -1)


def _cast_w(ws):
    return ws


@jax.jit
def kernel(x, edge_index, edge_attr, params):
    src = edge_index[0]
    dst = edge_index[1].reshape(NSUB, NCHUNK, CB)

    e_const = params["enc_edge"]["ln_b"][0]

    en = params["enc_node"]
    enc_w = []
    for l in en["mlp"]:
        enc_w += [l["w"], _r1(l["b"])]
    enc_w += [_r1(en["ln_g"]), _r1(en["ln_b"])]

    def msg_weights(t):
        lw = params["proc"][t]["edge_mlp"]
        w1 = lw[0]["w"]
        b1_eff = lw[0]["b"] + e_const * w1[D, :]
        return [w1[:D, :], _r1(b1_eff), lw[1]["w"], _r1(lw[1]["b"])]

    x0, ml, mr = _enc_call(x, _cast_w(enc_w + msg_weights(0)))

    sl, sr, cnt = _segsum_cnt(ml, mr, src, dst)
    recip = (1.0 / jnp.maximum(cnt, 1.0)).reshape(N, 1)

    for t in range(3):
        ls = params["proc"][t]["lin_self"]
        w = [ls["w"], _r1(ls["b"])] + msg_weights(t + 1)
        x0, ml, mr = _step_call(x0, sl, sr, recip, _cast_w(w))
        sl, sr = _segsum(ml, mr, src, dst)

    ls = params["proc"][3]["lin_self"]
    w = [ls["w"], _r1(ls["b"])]
    for l in params["dec"]:
        w += [l["w"], _r1(l["b"])]
    return _last_call(x0, sl, sr, recip, _cast_w(w))


# speedup vs baseline: 1.2426x; 1.0005x over previous
import functools

import jax
import jax.numpy as jnp
from jax import lax
from jax.experimental import pallas as pl
from jax.experimental.pallas import tpu as pltpu
from jax.experimental.pallas import tpu_sc as plsc

N = 10000
E = 160000
D = 256
HALF = 128
OUT_D = 3

NCORES = 2
NSUB = 16
EPT = E // NSUB
CB = 40
NCHUNK = EPT // CB
RING = 5
CNT_CB = 80
ZROWS = 16
NZB = N // ZROWS

BR = 2000
GRID = N // BR


def _bdot(x, w):
    return jnp.dot(x, w, preferred_element_type=jnp.float32)


def _msg(x, w1, b1, w2, b2):
    h = jnp.maximum(_bdot(x, w1) + b1, 0.0)
    return _bdot(h, w2) + b2


def _enc_body(x_ref, we1, be1, we2, be2, we3, be3, g_ref, b_ref,
              w1a, b1e, w2, b2, x0_ref, ml_ref, mr_ref):
    h = jnp.maximum(_bdot(x_ref[...], we1[...]) + be1[...], 0.0)
    h = jnp.maximum(_bdot(h, we2[...]) + be2[...], 0.0)
    h = _bdot(h, we3[...]) + be3[...]
    mu = jnp.mean(h, axis=1, keepdims=True)
    var = jnp.mean((h - mu) * (h - mu), axis=1, keepdims=True)
    x0 = (h - mu) / jnp.sqrt(var + 1e-5) * g_ref[...] + b_ref[...]
    x0_ref[...] = x0
    mm = _msg(x0, w1a[...], b1e[...], w2[...], b2[...])
    ml_ref[...] = mm[:, :HALF]
    mr_ref[...] = mm[:, HALF:]


def _step_body(x_ref, sl_ref, sr_ref, r_ref, ws, bs,
               w1a, b1e, w2, b2, xt_ref, ml_ref, mr_ref):
    aggr = jnp.concatenate([sl_ref[...], sr_ref[...]], axis=1) * r_ref[...]
    xt = _bdot(x_ref[...], ws[...]) + bs[...] + aggr
    xt_ref[...] = xt
    mm = _msg(xt, w1a[...], b1e[...], w2[...], b2[...])
    ml_ref[...] = mm[:, :HALF]
    mr_ref[...] = mm[:, HALF:]


def _last_body(x_ref, sl_ref, sr_ref, r_ref, ws, bs,
               wd1, bd1, wd2, bd2, wd3, bd3, o_ref):
    aggr = jnp.concatenate([sl_ref[...], sr_ref[...]], axis=1) * r_ref[...]
    xt = _bdot(x_ref[...], ws[...]) + bs[...] + aggr
    h = jnp.maximum(_bdot(xt, wd1[...]) + bd1[...], 0.0)
    h = jnp.maximum(_bdot(h, wd2[...]) + bd2[...], 0.0)
    o_ref[...] = _bdot(h, wd3[...]) + bd3[...]


def _row_spec(width):
    return pl.BlockSpec((BR, width), lambda i: (i, 0))


def _full_spec(shape):
    return pl.BlockSpec(shape, lambda i: tuple(0 for _ in shape))


def _wspec(a):
    return _full_spec(a.shape)


def _f32(shape):
    return jax.ShapeDtypeStruct(shape, jnp.float32)


def _enc_call(x, weights):
    in_specs = [_row_spec(D)] + [_wspec(w) for w in weights]
    return pl.pallas_call(
        _enc_body,
        grid=(GRID,),
        in_specs=in_specs,
        out_specs=[_row_spec(D), _row_spec(HALF), _row_spec(HALF)],
        out_shape=[_f32((N, D)), _f32((N, HALF)), _f32((N, HALF))],
    )(x, *weights)


def _step_call(x, sl, sr, recip, weights):
    in_specs = [_row_spec(D), _row_spec(HALF), _row_spec(HALF), _row_spec(1)]
    in_specs += [_wspec(w) for w in weights]
    return pl.pallas_call(
        _step_body,
        grid=(GRID,),
        in_specs=in_specs,
        out_specs=[_row_spec(D), _row_spec(HALF), _row_spec(HALF)],
        out_shape=[_f32((N, D)), _f32((N, HALF)), _f32((N, HALF))],
    )(x, sl, sr, recip, *weights)


def _last_call(x, sl, sr, recip, weights):
    in_specs = [_row_spec(D), _row_spec(HALF), _row_spec(HALF), _row_spec(1)]
    in_specs += [_wspec(w) for w in weights]
    return pl.pallas_call(
        _last_body,
        grid=(GRID,),
        in_specs=in_specs,
        out_specs=[_row_spec(OUT_D)],
        out_shape=[_f32((N, OUT_D))],
    )(x, sl, sr, recip, *weights)[0]


_MESH = plsc.VectorSubcoreMesh(
    core_axis_name="c", subcore_axis_name="s",
    num_cores=NCORES, num_subcores=NSUB)

_CNT_BLK = N // CNT_CB


def _half_pipeline(sid, m_hbm, src1, dst3, out_hbm, srcv, dstv,
                   rows, sems, ssems, zbuf, acc, zsem, isem, cnt_parts):
    pltpu.async_copy(src1.at[pl.ds(sid * EPT, EPT)], srcv, isem)

    nz = jnp.where(sid < NZB % NSUB, NZB // NSUB + 1, NZB // NSUB)

    def zbody(i, _):
        blk = sid + i * NSUB
        pltpu.async_copy(zbuf, acc.at[pl.ds(blk * ZROWS, ZROWS)], zsem)
        return ()
    lax.fori_loop(0, nz, zbody, (), unroll=False)

    if cnt_parts is not None:
        onesv, zc, acc_cnt, cnt_out, cwb = cnt_parts
        ncz = jnp.where(sid < _CNT_BLK % NSUB, _CNT_BLK // NSUB + 1,
                        _CNT_BLK // NSUB)

        def czbody(i, _):
            blk = sid + i * NSUB
            pltpu.sync_copy(zc, acc_cnt.at[pl.ds(blk * CNT_CB, CNT_CB)])
            return ()
        lax.fori_loop(0, ncz, czbody, (), unroll=False)

    def fire(j, b):
        pltpu.async_copy(dst3.at[sid, j], dstv.at[b], sems[b])
        pltpu.async_copy(m_hbm.at[srcv.at[pl.ds(j * CB, CB)]], rows[b],
                         sems[b])

    def wait(b):
        pltpu.make_async_copy(dst3.at[sid, 0], dstv.at[b], sems[b]).wait()
        pltpu.make_async_copy(m_hbm.at[srcv.at[pl.ds(0, CB)]], rows[b],
                              sems[b]).wait()

    def scat(j, b):
        pltpu.async_copy(rows[b], acc.at[dstv.at[b]], ssems[b], add=True)
        if cnt_parts is not None:
            pltpu.async_copy(cnt_parts[0], cnt_parts[2].at[dstv.at[b]],
                             ssems[b], add=True)

    def wait_scat(b):
        pltpu.make_async_copy(rows[b], acc.at[dstv.at[b]], ssems[b]).wait()
        if cnt_parts is not None:
            pltpu.make_async_copy(cnt_parts[0], cnt_parts[2].at[dstv.at[b]],
                                  ssems[b]).wait()

    pltpu.make_async_copy(src1.at[pl.ds(0, EPT)], srcv, isem).wait()
    for k in range(RING):
        fire(k, k)

    def zdrain(i, _):
        pltpu.make_async_copy(zbuf, acc.at[pl.ds(0, ZROWS)], zsem).wait()
        return ()
    lax.fori_loop(0, nz, zdrain, (), unroll=False)
    plsc.subcore_barrier()

    def body(i, _):
        j = RING * i
        for b in range(RING):
            wait(b)
            scat(j + b, b)

            @pl.when(j + b + RING < NCHUNK)
            def _():
                wait_scat(b)
                fire(j + b + RING, b)
        return ()
    lax.fori_loop(0, NCHUNK // RING, body, (), unroll=False)

    for b in range(RING):
        wait_scat(b)

    plsc.subcore_barrier()

    def wbody(i, _):
        blk = sid + i * NSUB
        pltpu.async_copy(acc.at[pl.ds(blk * ZROWS, ZROWS)],
                         out_hbm.at[pl.ds(blk * ZROWS, ZROWS)], zsem)
        return ()
    lax.fori_loop(0, nz, wbody, (), unroll=False)

    def wdrain(i, _):
        pltpu.make_async_copy(acc.at[pl.ds(0, ZROWS)],
                              out_hbm.at[pl.ds(0, ZROWS)], zsem).wait()
        return ()
    lax.fori_loop(0, nz, wdrain, (), unroll=False)

    if cnt_parts is not None:
        onesv, zc, acc_cnt, cnt_out, cwb = cnt_parts

        def cwbody(i, _):
            blk = sid + i * NSUB
            pltpu.sync_copy(acc_cnt.at[pl.ds(blk * CNT_CB, CNT_CB)], cwb)
            pltpu.sync_copy(cwb, cnt_out.at[pl.ds(blk * CNT_CB, CNT_CB)])
            return ()
        lax.fori_loop(0, ncz, cwbody, (), unroll=False)


def _fill(ref, length, value):
    v = jnp.full((16,), value, jnp.float32)
    for j in range(0, length - 15, 16):
        ref[pl.ds(j, 16)] = v
    if length % 16:
        ref[pl.ds(length - 16, 16)] = v


def _make_segsum(with_cnt):
    out_type = [_f32((N, HALF)), _f32((N, HALF))]
    scratch = [
        pltpu.VMEM((EPT,), jnp.int32),
        pltpu.VMEM((RING, CB), jnp.int32),
    ]
    scratch += [pltpu.VMEM((CB, HALF), jnp.float32)
                for _ in range(RING)]
    scratch += [
        pltpu.VMEM((ZROWS, HALF), jnp.float32),
        pltpu.VMEM_SHARED((N, HALF), jnp.float32),
    ]
    scratch += [pltpu.SemaphoreType.DMA for _ in range(RING)]
    scratch += [pltpu.SemaphoreType.DMA for _ in range(RING)]
    scratch += [
        pltpu.SemaphoreType.DMA,
        pltpu.SemaphoreType.DMA,
    ]
    if with_cnt:
        out_type.append(_f32((N,)))
        scratch += [
            pltpu.VMEM((CB,), jnp.float32),
            pltpu.VMEM((CNT_CB,), jnp.float32),
            pltpu.VMEM_SHARED((N,), jnp.float32),
            pltpu.VMEM((CNT_CB,), jnp.float32),
        ]

    def body(ml, mr, src3, dst3, sl_out, sr_out, *rest):
        if with_cnt:
            cnt_out = rest[0]
            rest = rest[1:]
        else:
            cnt_out = None
        srcv, dstv = rest[0], rest[1]
        rows = list(rest[2:2 + RING])
        zbuf, acc = rest[2 + RING], rest[3 + RING]
        sems = list(rest[4 + RING:4 + 2 * RING])
        ssems = list(rest[4 + 2 * RING:4 + 3 * RING])
        zsem, isem = rest[4 + 3 * RING], rest[5 + 3 * RING]
        if with_cnt:
            onesv, zc, acc_cnt, cwb = rest[6 + 3 * RING:]

        cid = lax.axis_index("c")
        sid = lax.axis_index("s")

        zero16 = jnp.zeros((16,), jnp.float32)

        def zfill(i, _):
            for j in range(HALF // 16):
                zbuf[i, pl.ds(j * 16, 16)] = zero16
            return ()
        lax.fori_loop(0, ZROWS, zfill, (), unroll=False)
        if with_cnt:
            _fill(onesv, CB, 1.0)
            _fill(zc, CNT_CB, 0.0)
            cnt_parts = (onesv, zc, acc_cnt, cnt_out, cwb)
        else:
            cnt_parts = None

        @pl.when(cid == 0)
        def _():
            _half_pipeline(sid, ml, src3, dst3, sl_out, srcv, dstv,
                           rows, sems, ssems, zbuf, acc, zsem, isem,
                           cnt_parts)

        @pl.when(cid == 1)
        def _():
            _half_pipeline(sid, mr, src3, dst3, sr_out, srcv, dstv,
                           rows, sems, ssems, zbuf, acc, zsem, isem, None)

    return pl.kernel(body, out_type=out_type, mesh=_MESH,
                     scratch_types=scratch)


_segsum_cnt = _make_segsum(True)
_segsum = _make_segsum(False)


def _r1(b):
    return b.reshape(1, -1)


@jax.jit
def kernel(x, edge_index, edge_attr, params):
    src = edge_index[0]
    dst = edge_index[1].reshape(NSUB, NCHUNK, CB)

    e_const = params["enc_edge"]["ln_b"][0]

    en = params["enc_node"]
    enc_w = []
    for l in en["mlp"]:
        enc_w += [l["w"], _r1(l["b"])]
    enc_w += [_r1(en["ln_g"]), _r1(en["ln_b"])]

    def msg_weights(t):
        lw = params["proc"][t]["edge_mlp"]
        w1 = lw[0]["w"]
        b1_eff = lw[0]["b"] + e_const * w1[D, :]
        return [w1[:D, :], _r1(b1_eff), lw[1]["w"], _r1(lw[1]["b"])]

    x0, ml, mr = _enc_call(x, enc_w + msg_weights(0))

    sl, sr, cnt = _segsum_cnt(ml, mr, src, dst)
    recip = (1.0 / jnp.maximum(cnt, 1.0)).reshape(N, 1)

    for t in range(3):
        ls = params["proc"][t]["lin_self"]
        w = [ls["w"], _r1(ls["b"])] + msg_weights(t + 1)
        x0, ml, mr = _step_call(x0, sl, sr, recip, w)
        sl, sr = _segsum(ml, mr, src, dst)

    ls = params["proc"][3]["lin_self"]
    w = [ls["w"], _r1(ls["b"])]
    for l in params["dec"]:
        w += [l["w"], _r1(l["b"])]
    return _last_call(x0, sl, sr, recip, w)
